# Initial kernel scaffold; baseline (speedup 1.0000x reference)
import jax
import jax.numpy as jnp
from jax.experimental import pallas as pl


def kernel(pred, label):
    raise NotImplementedError("write your pallas kernel here")


# speedup vs baseline: 33.3878x; 33.3878x over previous
import functools
import jax
import jax.numpy as jnp
from jax import lax
from jax.experimental import pallas as pl
from jax.experimental.pallas import tpu as pltpu

_GAMMA = 2.0
_ALPHA = 0.75
_NEG_RATIO = 3.0


def _focal_body(pred_ref, label_ref, out_ref):
    pred = pred_ref[...]
    label = label_ref[...]
    n = pred.shape[1]

    e = jnp.exp(-jnp.abs(pred))
    log1pe = jnp.log(1.0 + e)
    ls_pos = jnp.minimum(pred, 0.0) - log1pe
    ls_neg = jnp.minimum(-pred, 0.0) - log1pe
    p = jnp.where(pred >= 0.0, 1.0 / (1.0 + e), e / (1.0 + e))

    loss = -(label * ls_pos + (1.0 - label) * ls_neg)
    p_t = label * p + (1.0 - label) * (1.0 - p)
    m = 1.0 - p_t
    loss = loss * (m * m)
    alpha_factor = label * _ALPHA + (1.0 - label) * (1.0 - _ALPHA)
    loss = loss * alpha_factor

    fn = (p < 0.5) & (label == 1.0)
    fp = (p >= 0.5) & (label == 0.0)
    w = _ALPHA / (1.0 - _ALPHA)
    loss_w = jnp.where(fn | fp, loss * w, loss)

    pos = label > 0.0
    num_pos = jnp.sum(pos.astype(jnp.int32), axis=1, keepdims=True)
    num_neg = (_NEG_RATIO * num_pos.astype(jnp.float32)).astype(jnp.int32)
    k = jnp.minimum(num_neg, n - num_pos)

    bits = lax.bitcast_convert_type(loss, jnp.int32)
    key = jnp.where(pos, 0, bits + 1)

    def bit_step(i, res):
        cand = res | (1 << (30 - i))
        cnt = jnp.sum((key >= cand).astype(jnp.int32), axis=1, keepdims=True)
        return jnp.where(cnt >= k, cand, res)

    t = lax.fori_loop(0, 31, bit_step, jnp.zeros_like(k))

    gt = key > t
    eq = key == t
    n_gt = jnp.sum(gt.astype(jnp.int32), axis=1, keepdims=True)
    n_eq = jnp.sum(eq.astype(jnp.int32), axis=1, keepdims=True)
    sum_gt = jnp.sum(jnp.where(gt, loss_w, 0.0), axis=1, keepdims=True)
    sum_eq = jnp.sum(jnp.where(eq, loss_w, 0.0), axis=1, keepdims=True)
    take = jnp.clip(k - n_gt, 0, n_eq).astype(jnp.float32)
    eq_part = jnp.where(
        take > 0.0, take * sum_eq / jnp.maximum(n_eq, 1).astype(jnp.float32), 0.0
    )
    pos_sum = jnp.sum(jnp.where(pos, loss_w, 0.0), axis=1, keepdims=True)

    total = jnp.sum(pos_sum + sum_gt + eq_part)
    count = jnp.sum(num_pos + k).astype(jnp.float32)
    out_ref[...] = jnp.reshape(total / count, (1, 1))


@jax.jit
def kernel(pred, label):
    out = pl.pallas_call(
        _focal_body,
        out_shape=jax.ShapeDtypeStruct((1, 1), jnp.float32),
    )(pred, label)
    return out[0, 0]
